# Initial kernel scaffold; baseline (speedup 1.0000x reference)
import jax
import jax.numpy as jnp
from jax.experimental import pallas as pl


def kernel(embedded_words, action_vectors, W, b):
    raise NotImplementedError("write your pallas kernel here")


# speedup vs baseline: 7.4615x; 7.4615x over previous
import jax
import jax.numpy as jnp
from jax.experimental import pallas as pl
from jax.experimental.pallas import tpu as pltpu

_POINT = 11
_PAD_P = 16
_ACT = 4
_GROUPS = ((0, 4), (4, 9), (9, 10), (10, 11))

_BLK = 2048


def _body(x_ref, av_ref, w_ref, b_ref, out_ref):
    x = x_ref[...]
    av = av_ref[...]
    avn2 = jnp.sum(av * av, axis=0, keepdims=True)
    avs = av / jnp.maximum(jnp.sqrt(avn2), 1e-8)
    num = jnp.dot(x, avs, preferred_element_type=jnp.float32)
    ss = jnp.sum(x * x, axis=1, keepdims=True)
    sims = num / jnp.maximum(jnp.sqrt(ss), 1e-8)

    pooled = []
    for (s, e) in _GROUPS:
        m = sims[:, s:s + 1]
        for c in range(s + 1, e):
            m = jnp.maximum(m, sims[:, c:c + 1])
        pooled.append(m)

    logits = []
    for j in range(_ACT):
        l = b_ref[j]
        for k in range(_ACT):
            l = l + w_ref[j, k] * pooled[k]
        logits.append(l)

    m = jnp.maximum(jnp.maximum(logits[0], logits[1]),
                    jnp.maximum(logits[2], logits[3]))
    exps = [jnp.exp(l - m) for l in logits]
    tot = exps[0] + exps[1] + exps[2] + exps[3]
    out_ref[...] = jnp.concatenate([e / tot for e in exps], axis=1)


def kernel(embedded_words, action_vectors, W, b):
    B, D = embedded_words.shape
    av = jnp.pad(action_vectors[0], ((0, 0), (0, _PAD_P - _POINT)))
    grid = (B // _BLK,)
    return pl.pallas_call(
        _body,
        grid=grid,
        in_specs=[
            pl.BlockSpec((_BLK, D), lambda i: (i, 0)),
            pl.BlockSpec((D, _PAD_P), lambda i: (0, 0)),
            pl.BlockSpec(memory_space=pltpu.SMEM),
            pl.BlockSpec(memory_space=pltpu.SMEM),
        ],
        out_specs=pl.BlockSpec((_BLK, _ACT), lambda i: (i, 0)),
        out_shape=jax.ShapeDtypeStruct((B, _ACT), jnp.float32),
    )(embedded_words, av, W, b)


# speedup vs baseline: 41.3280x; 5.5388x over previous
import jax
import jax.numpy as jnp
from jax import lax
from jax.experimental import pallas as pl
from jax.experimental.pallas import tpu as pltpu

_POINT = 11
_PAD_P = 16
_ACT = 4
_GROUPS = ((0, 4), (4, 9), (9, 10), (10, 11))

_BLK = 2048


def _body(x_ref, av_ref, w_ref, b_ref, out_ref):
    x = x_ref[...]
    av = av_ref[...]
    avn2 = jnp.sum(av * av, axis=0, keepdims=True)
    avs = av / jnp.maximum(jnp.sqrt(avn2), 1e-8)
    numT = lax.dot_general(avs, x, (((0,), (1,)), ((), ())),
                           preferred_element_type=jnp.float32)
    ones = jnp.ones((1, x_ref.shape[1]), jnp.float32)
    ssT = lax.dot_general(ones, x * x, (((1,), (1,)), ((), ())),
                          preferred_element_type=jnp.float32)
    simsT = numT / jnp.maximum(jnp.sqrt(ssT), 1e-8)

    pooled = []
    for (s, e) in _GROUPS:
        pooled.append(jnp.max(simsT[s:e], axis=0, keepdims=True))

    logits = []
    for j in range(_ACT):
        l = b_ref[j]
        for k in range(_ACT):
            l = l + w_ref[j, k] * pooled[k]
        logits.append(l)

    m = jnp.maximum(jnp.maximum(logits[0], logits[1]),
                    jnp.maximum(logits[2], logits[3]))
    exps = [jnp.exp(l - m) for l in logits]
    tot = exps[0] + exps[1] + exps[2] + exps[3]
    out_ref[...] = jnp.concatenate([e / tot for e in exps], axis=0)


def kernel(embedded_words, action_vectors, W, b):
    B, D = embedded_words.shape
    av = jnp.pad(action_vectors[0], ((0, 0), (0, _PAD_P - _POINT)))
    grid = (B // _BLK,)
    outT = pl.pallas_call(
        _body,
        grid=grid,
        in_specs=[
            pl.BlockSpec((_BLK, D), lambda i: (i, 0)),
            pl.BlockSpec((D, _PAD_P), lambda i: (0, 0)),
            pl.BlockSpec(memory_space=pltpu.SMEM),
            pl.BlockSpec(memory_space=pltpu.SMEM),
        ],
        out_specs=pl.BlockSpec((_ACT, _BLK), lambda i: (0, i)),
        out_shape=jax.ShapeDtypeStruct((_ACT, B), jnp.float32),
    )(embedded_words, av, W, b)
    return outT.T
